# Initial kernel scaffold; baseline (speedup 1.0000x reference)
import jax
import jax.numpy as jnp
from jax.experimental import pallas as pl


def kernel(node_ids, edge_index, emb, W0, b0, Wr0, br0, W1, b1, Wr1, br1):
    raise NotImplementedError("write your pallas kernel here")


# speedup vs baseline: 9.5304x; 9.5304x over previous
import functools

import jax
import jax.numpy as jnp
from jax import lax
from jax.experimental import pallas as pl
from jax.experimental.pallas import tpu as pltpu
from jax.experimental.pallas import tpu_sc as plsc

N = 10000
D = 128
E = 320000
VOCAB = 343
VPAD = 384
NP = 10240
BN = 1024
G = NP // BN
NC = 2
NS = 16
NW = NC * NS
EPW = E // NW
CH = 125
NCH = EPW // CH
RPT = NP // NS
CHZ = 128
DH = 64

_mesh = plsc.VectorSubcoreMesh(core_axis_name="c", subcore_axis_name="s")


def _dot(a, b):
    return lax.dot_general(a, b, (((1,), (0,)), ((), ())),
                           precision=lax.Precision.HIGHEST,
                           preferred_element_type=jnp.float32)


def _deg_body(src_hbm, dst_hbm, hs_out, hd_out, src_v, dst_v, hs_v, hd_v):
    c = lax.axis_index("c")
    s = lax.axis_index("s")
    w = s * NC + c
    pltpu.sync_copy(src_hbm.at[w], src_v)
    pltpu.sync_copy(dst_hbm.at[w], dst_v)

    zeros16 = jnp.zeros((16,), jnp.float32)
    ones16 = jnp.ones((16,), jnp.float32)

    def zero_step(i, carry):
        hs_v[pl.ds(i * 16, 16)] = zeros16
        hd_v[pl.ds(i * 16, 16)] = zeros16
        return carry

    lax.fori_loop(0, N // 16, zero_step, 0)

    def acc_step(i, carry):
        si = src_v[pl.ds(i * 16, 16)]
        di = dst_v[pl.ds(i * 16, 16)]
        plsc.addupdate_scatter(hs_v, [si], ones16)
        plsc.addupdate_scatter(hd_v, [di], ones16)
        return carry

    lax.fori_loop(0, EPW // 16, acc_step, 0)

    pltpu.sync_copy(hs_v, hs_out.at[w, 0])
    pltpu.sync_copy(hd_v, hd_out.at[w, 0])


_deg_kernel = functools.partial(
    pl.kernel,
    out_type=(jax.ShapeDtypeStruct((NW, 1, N), jnp.float32),
              jax.ShapeDtypeStruct((NW, 1, N), jnp.float32)),
    mesh=_mesh,
    compiler_params=pltpu.CompilerParams(needs_layout_passes=False,
                                         use_tc_tiling_on_sc=False),
    scratch_types=[
        pltpu.VMEM((EPW,), jnp.int32),
        pltpu.VMEM((EPW,), jnp.int32),
        pltpu.VMEM((N,), jnp.float32),
        pltpu.VMEM((N,), jnp.float32),
    ],
)(_deg_body)


def _agg_body(y_hbm, src_hbm, dst_hbm, out_hbm,
              src_v, dst_v, rows_v, zer_v, acc_sh, sem):
    c = lax.axis_index("c")
    s = lax.axis_index("s")
    w = s * NC + c
    pltpu.sync_copy(src_hbm.at[w], src_v)
    pltpu.sync_copy(dst_hbm.at[w], dst_v)

    zeros16 = jnp.zeros((16,), jnp.float32)

    def zero_step(i, carry):
        r = i // (DH // 16)
        col = (i % (DH // 16)) * 16
        zer_v[r, pl.ds(col, 16)] = zeros16
        return carry

    lax.fori_loop(0, CHZ * (DH // 16), zero_step, 0)

    for k in range(RPT // CHZ):
        pltpu.sync_copy(zer_v, acc_sh.at[pl.ds(s * RPT + k * CHZ, CHZ)])
    plsc.subcore_barrier()

    def chunk_step(j, carry):
        pltpu.async_copy(y_hbm.at[src_v.at[j]], rows_v, sem).wait()
        pltpu.sync_copy(rows_v, acc_sh.at[dst_v.at[j]], add=True)
        return carry

    lax.fori_loop(0, NCH, chunk_step, 0)
    plsc.subcore_barrier()

    for k in range(RPT // CHZ):
        r0 = s * RPT + k * CHZ
        pltpu.sync_copy(acc_sh.at[pl.ds(r0, CHZ)],
                        out_hbm.at[c, pl.ds(r0, CHZ)])


_agg_kernel = functools.partial(
    pl.kernel,
    out_type=jax.ShapeDtypeStruct((NC, NP, DH), jnp.float32),
    mesh=_mesh,
    compiler_params=pltpu.CompilerParams(needs_layout_passes=False,
                                         use_tc_tiling_on_sc=False),
    scratch_types=[
        pltpu.VMEM((NCH, CH), jnp.int32),
        pltpu.VMEM((NCH, CH), jnp.int32),
        pltpu.VMEM((CH, DH), jnp.float32),
        pltpu.VMEM((CHZ, DH), jnp.float32),
        pltpu.VMEM_SHARED((NP, DH), jnp.float32),
        pltpu.SemaphoreType.DMA,
    ],
)(_agg_body)


def _prep_body(hs_ref, hd_ref, emb_ref, w0_ref, wr0_ref, br0_ref,
               norms_ref, t0_ref, tr0_ref):
    degs = jnp.sum(hs_ref[...], axis=(0, 1))
    degd = jnp.sum(hd_ref[...], axis=(0, 1))
    deg = jnp.stack([degs, degd], axis=0)
    norms_ref[...] = lax.rsqrt(jnp.clip(deg, 1.0, None))
    embf = emb_ref[...]
    t0_ref[...] = _dot(embf, w0_ref[...])
    tr0_ref[...] = jax.nn.relu(_dot(embf, wr0_ref[...]) + br0_ref[0:1, :])


def _prep_call(hs, hd, emb_pad, w0, wr0, br0p):
    return pl.pallas_call(
        _prep_body,
        out_shape=(
            jax.ShapeDtypeStruct((2, N), jnp.float32),
            jax.ShapeDtypeStruct((VPAD, D), jnp.float32),
            jax.ShapeDtypeStruct((VPAD, D), jnp.float32),
        ),
    )(hs, hd, emb_pad, w0, wr0, br0p)


def _l1_body(nid_ref, ns_ref, t0_ref, tr0_ref, y0_ref, res0_ref):
    nid = nid_ref[0, 0, :]
    onehot = (nid[:, None] ==
              lax.broadcasted_iota(jnp.int32, (BN, VPAD), 1)
              ).astype(jnp.float32)
    ns = ns_ref[0, 0, :]
    y0_ref[...] = _dot(onehot, t0_ref[...]) * ns[:, None]
    res0_ref[...] = _dot(onehot, tr0_ref[...])


def _l1_call(nid3, ns3, t0, tr0):
    blk3 = pl.BlockSpec((1, 1, BN), lambda i: (i, 0, 0))
    full = pl.BlockSpec((VPAD, D), lambda i: (0, 0))
    rows = pl.BlockSpec((BN, D), lambda i: (i, 0))
    return pl.pallas_call(
        _l1_body,
        grid=(G,),
        in_specs=[blk3, blk3, full, full],
        out_specs=(rows, rows),
        out_shape=(
            jax.ShapeDtypeStruct((NP, D), jnp.float32),
            jax.ShapeDtypeStruct((NP, D), jnp.float32),
        ),
    )(nid3, ns3, t0, tr0)


def _mid_body(a0_ref, a1_ref, nd_ref, ns_ref, res0_ref, b0_ref,
              w1_ref, wr1_ref, br1_ref, y1_ref, res1_ref):
    nd = nd_ref[0, 0, :]
    ns = ns_ref[0, 0, :]
    agg = a0_ref[...] + a1_ref[...]
    h1 = jax.nn.relu(agg * nd[:, None] + b0_ref[0:1, :]) + res0_ref[...]
    y1_ref[...] = _dot(h1, w1_ref[...]) * ns[:, None]
    res1_ref[...] = jax.nn.relu(_dot(h1, wr1_ref[...]) + br1_ref[0:1, :])


def _mid_call(a0, a1, nd3, ns3, res0, b0p, w1, wr1, br1p):
    blk3 = pl.BlockSpec((1, 1, BN), lambda i: (i, 0, 0))
    rows = pl.BlockSpec((BN, D), lambda i: (i, 0))
    wfull = pl.BlockSpec((D, D), lambda i: (0, 0))
    bfull = pl.BlockSpec((8, D), lambda i: (0, 0))
    return pl.pallas_call(
        _mid_body,
        grid=(G,),
        in_specs=[rows, rows, blk3, blk3, rows, bfull, wfull, wfull, bfull],
        out_specs=(rows, rows),
        out_shape=(
            jax.ShapeDtypeStruct((NP, D), jnp.float32),
            jax.ShapeDtypeStruct((NP, D), jnp.float32),
        ),
    )(a0, a1, nd3, ns3, res0, b0p, w1, wr1, br1p)


def _fin_body(a0_ref, a1_ref, nd_ref, res1_ref, b1_ref, out_ref):
    nd = nd_ref[0, 0, :]
    agg = a0_ref[...] + a1_ref[...]
    out_ref[...] = (jax.nn.relu(agg * nd[:, None] + b1_ref[0:1, :])
                    + res1_ref[...])


def _fin_call(a0, a1, nd3, res1, b1p):
    blk3 = pl.BlockSpec((1, 1, BN), lambda i: (i, 0, 0))
    rows = pl.BlockSpec((BN, D), lambda i: (i, 0))
    bfull = pl.BlockSpec((8, D), lambda i: (0, 0))
    return pl.pallas_call(
        _fin_body,
        grid=(G,),
        in_specs=[rows, rows, blk3, rows, bfull],
        out_specs=rows,
        out_shape=jax.ShapeDtypeStruct((NP, D), jnp.float32),
    )(a0, a1, nd3, res1, b1p)


def kernel(node_ids, edge_index, emb, W0, b0, Wr0, br0, W1, b1, Wr1, br1):
    src = edge_index[0].astype(jnp.int32)
    dst = edge_index[1].astype(jnp.int32)
    srcr = src.reshape(NW, NCH, CH)
    dstr = dst.reshape(NW, NCH, CH)
    srcf = src.reshape(NW, EPW)
    dstf = dst.reshape(NW, EPW)

    nid = node_ids.astype(jnp.int32)
    nid3 = jnp.pad(nid, (0, NP - N)).reshape(G, 1, BN)
    emb_pad = jnp.zeros((VPAD, D), jnp.float32).at[:VOCAB].set(emb)
    b0p = jnp.broadcast_to(b0[None, :], (8, D))
    br0p = jnp.broadcast_to(br0[None, :], (8, D))
    b1p = jnp.broadcast_to(b1[None, :], (8, D))
    br1p = jnp.broadcast_to(br1[None, :], (8, D))

    hs, hd = _deg_kernel(srcf, dstf)
    norms, t0, tr0 = _prep_call(hs, hd, emb_pad, W0, Wr0, br0p)
    norms_p = jnp.pad(norms, ((0, 0), (0, NP - N)))
    ns3 = norms_p[0].reshape(G, 1, BN)
    nd3 = norms_p[1].reshape(G, 1, BN)

    y0, res0 = _l1_call(nid3, ns3, t0, tr0)

    def agg(y):
        lo = _agg_kernel(y[:, :DH], srcr, dstr)
        hi = _agg_kernel(y[:, DH:], srcr, dstr)
        a0 = jnp.concatenate([lo[0], hi[0]], axis=1)
        a1 = jnp.concatenate([lo[1], hi[1]], axis=1)
        return a0, a1

    a0, a1 = agg(y0)
    y1, res1 = _mid_call(a0, a1, nd3, ns3, res0, b0p, W1, Wr1, br1p)
    a0, a1 = agg(y1)
    h2 = _fin_call(a0, a1, nd3, res1, b1p)
    return h2[:N]


# speedup vs baseline: 14.5305x; 1.5246x over previous
import functools

import jax
import jax.numpy as jnp
from jax import lax
from jax.experimental import pallas as pl
from jax.experimental.pallas import tpu as pltpu
from jax.experimental.pallas import tpu_sc as plsc

N = 10000
D = 128
E = 320000
VOCAB = 343
VPAD = 384
NP = 10240
BN = 1024
G = NP // BN
NC = 2
NS = 16
NW = NC * NS
EPW = E // NW
CH = 125
NCH = EPW // CH
RPT = NP // NS
CHZ = 128
DH = 64

_mesh = plsc.VectorSubcoreMesh(core_axis_name="c", subcore_axis_name="s")


def _dot(a, b):
    return lax.dot_general(a, b, (((1,), (0,)), ((), ())),
                           precision=lax.Precision.HIGHEST,
                           preferred_element_type=jnp.float32)


def _deg_body(src_hbm, dst_hbm, hs_out, hd_out, src_v, dst_v, hs_v, hd_v):
    c = lax.axis_index("c")
    s = lax.axis_index("s")
    w = s * NC + c
    pltpu.sync_copy(src_hbm.at[w], src_v)
    pltpu.sync_copy(dst_hbm.at[w], dst_v)

    zeros16 = jnp.zeros((16,), jnp.float32)
    ones16 = jnp.ones((16,), jnp.float32)

    def zero_step(i, carry):
        hs_v[pl.ds(i * 16, 16)] = zeros16
        hd_v[pl.ds(i * 16, 16)] = zeros16
        return carry

    lax.fori_loop(0, N // 16, zero_step, 0)

    def acc_step(i, carry):
        si = src_v[pl.ds(i * 16, 16)]
        di = dst_v[pl.ds(i * 16, 16)]
        plsc.addupdate_scatter(hs_v, [si], ones16)
        plsc.addupdate_scatter(hd_v, [di], ones16)
        return carry

    lax.fori_loop(0, EPW // 16, acc_step, 0)

    pltpu.sync_copy(hs_v, hs_out.at[w, 0])
    pltpu.sync_copy(hd_v, hd_out.at[w, 0])


_deg_kernel = functools.partial(
    pl.kernel,
    out_type=(jax.ShapeDtypeStruct((NW, 1, N), jnp.float32),
              jax.ShapeDtypeStruct((NW, 1, N), jnp.float32)),
    mesh=_mesh,
    compiler_params=pltpu.CompilerParams(needs_layout_passes=False,
                                         use_tc_tiling_on_sc=False),
    scratch_types=[
        pltpu.VMEM((EPW,), jnp.int32),
        pltpu.VMEM((EPW,), jnp.int32),
        pltpu.VMEM((N,), jnp.float32),
        pltpu.VMEM((N,), jnp.float32),
    ],
)(_deg_body)


def _agg_body(ylo_hbm, yhi_hbm, src_hbm, dst_hbm, out_hbm,
              src_v, dst_v, rows_a, rows_b, zer_v, acc_sh, sem_a, sem_b):
    c = lax.axis_index("c")
    s = lax.axis_index("s")
    w = s * NC + c
    pltpu.sync_copy(src_hbm.at[w], src_v)
    pltpu.sync_copy(dst_hbm.at[w], dst_v)

    zeros16 = jnp.zeros((16,), jnp.float32)

    def zero_step(i, carry):
        r = i // (DH // 16)
        col = (i % (DH // 16)) * 16
        zer_v[r, pl.ds(col, 16)] = zeros16
        return carry

    lax.fori_loop(0, CHZ * (DH // 16), zero_step, 0)

    def zero_acc():
        for k in range(RPT // CHZ):
            pltpu.sync_copy(zer_v, acc_sh.at[pl.ds(s * RPT + k * CHZ, CHZ)])

    zero_acc()
    plsc.subcore_barrier()

    for h, y_hbm in enumerate((ylo_hbm, yhi_hbm)):
        pltpu.async_copy(y_hbm.at[src_v.at[0]], rows_a, sem_a)

        def step(jj, carry):
            j = jj * 2
            pltpu.async_copy(y_hbm.at[src_v.at[j + 1]], rows_b, sem_b)
            pltpu.make_async_copy(y_hbm.at[src_v.at[j]], rows_a, sem_a).wait()
            pltpu.sync_copy(rows_a, acc_sh.at[dst_v.at[j]], add=True)

            @pl.when(j + 2 < NCH)
            def _():
                pltpu.async_copy(y_hbm.at[src_v.at[j + 2]], rows_a, sem_a)

            pltpu.make_async_copy(y_hbm.at[src_v.at[j + 1]], rows_b,
                                  sem_b).wait()
            pltpu.sync_copy(rows_b, acc_sh.at[dst_v.at[j + 1]], add=True)
            return carry

        lax.fori_loop(0, NCH // 2, step, 0)
        plsc.subcore_barrier()

        for k in range(RPT // CHZ):
            r0 = s * RPT + k * CHZ
            pltpu.sync_copy(acc_sh.at[pl.ds(r0, CHZ)],
                            out_hbm.at[h, c, pl.ds(r0, CHZ)])
        if h == 0:
            zero_acc()
            plsc.subcore_barrier()


_agg_kernel = functools.partial(
    pl.kernel,
    out_type=jax.ShapeDtypeStruct((2, NC, NP, DH), jnp.float32),
    mesh=_mesh,
    compiler_params=pltpu.CompilerParams(needs_layout_passes=False,
                                         use_tc_tiling_on_sc=False),
    scratch_types=[
        pltpu.VMEM((NCH, CH), jnp.int32),
        pltpu.VMEM((NCH, CH), jnp.int32),
        pltpu.VMEM((CH, DH), jnp.float32),
        pltpu.VMEM((CH, DH), jnp.float32),
        pltpu.VMEM((CHZ, DH), jnp.float32),
        pltpu.VMEM_SHARED((NP, DH), jnp.float32),
        pltpu.SemaphoreType.DMA,
        pltpu.SemaphoreType.DMA,
    ],
)(_agg_body)


def _prep_body(hs_ref, hd_ref, emb_ref, w0_ref, wr0_ref, br0_ref,
               norms_ref, t0_ref, tr0_ref):
    degs = jnp.sum(hs_ref[...], axis=(0, 1))
    degd = jnp.sum(hd_ref[...], axis=(0, 1))
    deg = jnp.stack([degs, degd], axis=0)
    norms_ref[...] = lax.rsqrt(jnp.clip(deg, 1.0, None))
    embf = emb_ref[...]
    t0_ref[...] = _dot(embf, w0_ref[...])
    tr0_ref[...] = jax.nn.relu(_dot(embf, wr0_ref[...]) + br0_ref[0:1, :])


def _prep_call(hs, hd, emb_pad, w0, wr0, br0p):
    return pl.pallas_call(
        _prep_body,
        out_shape=(
            jax.ShapeDtypeStruct((2, N), jnp.float32),
            jax.ShapeDtypeStruct((VPAD, D), jnp.float32),
            jax.ShapeDtypeStruct((VPAD, D), jnp.float32),
        ),
    )(hs, hd, emb_pad, w0, wr0, br0p)


def _l1_body(nid_ref, ns_ref, t0_ref, tr0_ref, ylo_ref, yhi_ref, res0_ref):
    nid = nid_ref[0, 0, :]
    onehot = (nid[:, None] ==
              lax.broadcasted_iota(jnp.int32, (BN, VPAD), 1)
              ).astype(jnp.float32)
    ns = ns_ref[0, 0, :]
    y0 = _dot(onehot, t0_ref[...]) * ns[:, None]
    ylo_ref[...] = y0[:, :DH]
    yhi_ref[...] = y0[:, DH:]
    res0_ref[...] = _dot(onehot, tr0_ref[...])


def _l1_call(nid3, ns3, t0, tr0):
    blk3 = pl.BlockSpec((1, 1, BN), lambda i: (i, 0, 0))
    full = pl.BlockSpec((VPAD, D), lambda i: (0, 0))
    rows = pl.BlockSpec((BN, D), lambda i: (i, 0))
    half = pl.BlockSpec((BN, DH), lambda i: (i, 0))
    return pl.pallas_call(
        _l1_body,
        grid=(G,),
        in_specs=[blk3, blk3, full, full],
        out_specs=(half, half, rows),
        out_shape=(
            jax.ShapeDtypeStruct((NP, DH), jnp.float32),
            jax.ShapeDtypeStruct((NP, DH), jnp.float32),
            jax.ShapeDtypeStruct((NP, D), jnp.float32),
        ),
    )(nid3, ns3, t0, tr0)


def _agg_specs():
    return [pl.BlockSpec((1, 1, BN, DH), lambda i, h=h, c=c: (h, c, i, 0))
            for h in (0, 1) for c in (0, 1)]


def _combine(a_lo0, a_lo1, a_hi0, a_hi1):
    lo = a_lo0[0, 0] + a_lo1[0, 0]
    hi = a_hi0[0, 0] + a_hi1[0, 0]
    return jnp.concatenate([lo, hi], axis=-1)


def _mid_body(alo0_ref, alo1_ref, ahi0_ref, ahi1_ref, nd_ref, ns_ref,
              res0_ref, b0_ref, w1_ref, wr1_ref, br1_ref,
              ylo_ref, yhi_ref, res1_ref):
    nd = nd_ref[0, 0, :]
    ns = ns_ref[0, 0, :]
    agg = _combine(alo0_ref[...], alo1_ref[...],
                   ahi0_ref[...], ahi1_ref[...])
    h1 = jax.nn.relu(agg * nd[:, None] + b0_ref[0:1, :]) + res0_ref[...]
    y1 = _dot(h1, w1_ref[...]) * ns[:, None]
    ylo_ref[...] = y1[:, :DH]
    yhi_ref[...] = y1[:, DH:]
    res1_ref[...] = jax.nn.relu(_dot(h1, wr1_ref[...]) + br1_ref[0:1, :])


def _mid_call(aggout, nd3, ns3, res0, b0p, w1, wr1, br1p):
    s00, s01, s10, s11 = _agg_specs()
    blk3 = pl.BlockSpec((1, 1, BN), lambda i: (i, 0, 0))
    rows = pl.BlockSpec((BN, D), lambda i: (i, 0))
    half = pl.BlockSpec((BN, DH), lambda i: (i, 0))
    wfull = pl.BlockSpec((D, D), lambda i: (0, 0))
    bfull = pl.BlockSpec((8, D), lambda i: (0, 0))
    return pl.pallas_call(
        _mid_body,
        grid=(G,),
        in_specs=[s00, s01, s10, s11, blk3, blk3, rows, bfull,
                  wfull, wfull, bfull],
        out_specs=(half, half, rows),
        out_shape=(
            jax.ShapeDtypeStruct((NP, DH), jnp.float32),
            jax.ShapeDtypeStruct((NP, DH), jnp.float32),
            jax.ShapeDtypeStruct((NP, D), jnp.float32),
        ),
    )(aggout, aggout, aggout, aggout, nd3, ns3, res0, b0p, w1, wr1, br1p)


def _fin_body(alo0_ref, alo1_ref, ahi0_ref, ahi1_ref, nd_ref, res1_ref,
              b1_ref, out_ref):
    nd = nd_ref[0, 0, :]
    agg = _combine(alo0_ref[...], alo1_ref[...],
                   ahi0_ref[...], ahi1_ref[...])
    out_ref[...] = (jax.nn.relu(agg * nd[:, None] + b1_ref[0:1, :])
                    + res1_ref[...])


def _fin_call(aggout, nd3, res1, b1p):
    s00, s01, s10, s11 = _agg_specs()
    blk3 = pl.BlockSpec((1, 1, BN), lambda i: (i, 0, 0))
    rows = pl.BlockSpec((BN, D), lambda i: (i, 0))
    bfull = pl.BlockSpec((8, D), lambda i: (0, 0))
    return pl.pallas_call(
        _fin_body,
        grid=(G,),
        in_specs=[s00, s01, s10, s11, blk3, rows, bfull],
        out_specs=rows,
        out_shape=jax.ShapeDtypeStruct((NP, D), jnp.float32),
    )(aggout, aggout, aggout, aggout, nd3, res1, b1p)


def kernel(node_ids, edge_index, emb, W0, b0, Wr0, br0, W1, b1, Wr1, br1):
    src = edge_index[0].astype(jnp.int32)
    dst = edge_index[1].astype(jnp.int32)
    srcr = src.reshape(NW, NCH, CH)
    dstr = dst.reshape(NW, NCH, CH)
    srcf = src.reshape(NW, EPW)
    dstf = dst.reshape(NW, EPW)

    nid = node_ids.astype(jnp.int32)
    nid3 = jnp.pad(nid, (0, NP - N)).reshape(G, 1, BN)
    emb_pad = jnp.zeros((VPAD, D), jnp.float32).at[:VOCAB].set(emb)
    b0p = jnp.broadcast_to(b0[None, :], (8, D))
    br0p = jnp.broadcast_to(br0[None, :], (8, D))
    b1p = jnp.broadcast_to(b1[None, :], (8, D))
    br1p = jnp.broadcast_to(br1[None, :], (8, D))

    hs, hd = _deg_kernel(srcf, dstf)
    norms, t0, tr0 = _prep_call(hs, hd, emb_pad, W0, Wr0, br0p)
    norms_p = jnp.pad(norms, ((0, 0), (0, NP - N)))
    ns3 = norms_p[0].reshape(G, 1, BN)
    nd3 = norms_p[1].reshape(G, 1, BN)

    y0lo, y0hi, res0 = _l1_call(nid3, ns3, t0, tr0)
    agg1 = _agg_kernel(y0lo, y0hi, srcr, dstr)
    y1lo, y1hi, res1 = _mid_call(agg1, nd3, ns3, res0, b0p, W1, Wr1, br1p)
    agg2 = _agg_kernel(y1lo, y1hi, srcr, dstr)
    h2 = _fin_call(agg2, nd3, res1, b1p)
    return h2[:N]


# speedup vs baseline: 16.1928x; 1.1144x over previous
import functools

import jax
import jax.numpy as jnp
from jax import lax
from jax.experimental import pallas as pl
from jax.experimental.pallas import tpu as pltpu
from jax.experimental.pallas import tpu_sc as plsc

N = 10000
D = 128
E = 320000
VOCAB = 343
VPAD = 384
NP = 10240
BN = 1024
G = NP // BN
NC = 2
NS = 16
NW = NC * NS
EPW = E // NW
CH = 125
NCH = EPW // CH
RPT = NP // NS
CHZ = 128
DH = 64

_mesh = plsc.VectorSubcoreMesh(core_axis_name="c", subcore_axis_name="s")


def _dot(a, b):
    return lax.dot_general(a, b, (((1,), (0,)), ((), ())),
                           precision=lax.Precision.HIGHEST,
                           preferred_element_type=jnp.float32)


def _deg_body(src_hbm, dst_hbm, hs_out, hd_out, src_v, dst_v, hs_v, hd_v):
    c = lax.axis_index("c")
    s = lax.axis_index("s")
    w = s * NC + c
    pltpu.sync_copy(src_hbm.at[w], src_v)
    pltpu.sync_copy(dst_hbm.at[w], dst_v)

    zeros16 = jnp.zeros((16,), jnp.float32)
    ones16 = jnp.ones((16,), jnp.float32)

    def zero_step(i, carry):
        hs_v[pl.ds(i * 16, 16)] = zeros16
        hd_v[pl.ds(i * 16, 16)] = zeros16
        return carry

    lax.fori_loop(0, N // 16, zero_step, 0)

    def acc_step(i, carry):
        si = src_v[pl.ds(i * 16, 16)]
        di = dst_v[pl.ds(i * 16, 16)]
        plsc.addupdate_scatter(hs_v, [si], ones16)
        plsc.addupdate_scatter(hd_v, [di], ones16)
        return carry

    lax.fori_loop(0, EPW // 16, acc_step, 0)

    pltpu.sync_copy(hs_v, hs_out.at[w, 0])
    pltpu.sync_copy(hd_v, hd_out.at[w, 0])


_deg_kernel = functools.partial(
    pl.kernel,
    out_type=(jax.ShapeDtypeStruct((NW, 1, N), jnp.float32),
              jax.ShapeDtypeStruct((NW, 1, N), jnp.float32)),
    mesh=_mesh,
    compiler_params=pltpu.CompilerParams(needs_layout_passes=False,
                                         use_tc_tiling_on_sc=False),
    scratch_types=[
        pltpu.VMEM((EPW,), jnp.int32),
        pltpu.VMEM((EPW,), jnp.int32),
        pltpu.VMEM((N,), jnp.float32),
        pltpu.VMEM((N,), jnp.float32),
    ],
)(_deg_body)


def _agg_body(ylo_hbm, yhi_hbm, src_hbm, dst_hbm, out_hbm,
              src_v, dst_v, rows_a, rows_b, rows_c, rows_d, zer_v, acc_sh,
              gsem_a, gsem_b, gsem_c, gsem_d,
              ssem_a, ssem_b, ssem_c, ssem_d):
    c = lax.axis_index("c")
    s = lax.axis_index("s")
    w = s * NC + c
    pltpu.sync_copy(src_hbm.at[w], src_v)
    pltpu.sync_copy(dst_hbm.at[w], dst_v)

    zeros16 = jnp.zeros((16,), jnp.float32)

    def zero_step(i, carry):
        r = i // (DH // 16)
        col = (i % (DH // 16)) * 16
        zer_v[r, pl.ds(col, 16)] = zeros16
        return carry

    lax.fori_loop(0, CHZ * (DH // 16), zero_step, 0)

    def zero_acc():
        for k in range(RPT // CHZ):
            pltpu.sync_copy(zer_v, acc_sh.at[pl.ds(s * RPT + k * CHZ, CHZ)])

    zero_acc()
    plsc.subcore_barrier()

    for h, y_hbm in enumerate((ylo_hbm, yhi_hbm)):
        bufs = (rows_a, rows_b, rows_c, rows_d)
        gsem = (gsem_a, gsem_b, gsem_c, gsem_d)
        ssem = (ssem_a, ssem_b, ssem_c, ssem_d)

        def issue_g(b, j):
            pltpu.async_copy(y_hbm.at[src_v.at[j]], bufs[b], gsem[b])

        def wait_g(b, j):
            pltpu.make_async_copy(y_hbm.at[src_v.at[j]], bufs[b],
                                  gsem[b]).wait()

        def issue_s(b, j):
            pltpu.async_copy(bufs[b], acc_sh.at[dst_v.at[j]], ssem[b],
                             add=True)

        def wait_s(b, j):
            pltpu.make_async_copy(bufs[b], acc_sh.at[dst_v.at[j]],
                                  ssem[b]).wait()

        for b in range(3):
            issue_g(b, b)

        def step(jj, carry):
            for k in range(4):
                j = jj * 4 + k
                wait_g(k, j)
                issue_s(k, j)
                kp = (k - 1) % 4
                if k == 0:
                    @pl.when(j >= 1)
                    def _():
                        wait_s(kp, j - 1)

                    issue_g(kp, j + 3)
                else:
                    wait_s(kp, j - 1)

                    @pl.when(j + 3 < NCH)
                    def _():
                        issue_g(kp, j + 3)
            return carry

        lax.fori_loop(0, NCH // 4, step, 0)
        wait_s((NCH - 1) % 4, NCH - 1)
        plsc.subcore_barrier()

        for k in range(RPT // CHZ):
            r0 = s * RPT + k * CHZ
            pltpu.sync_copy(acc_sh.at[pl.ds(r0, CHZ)],
                            out_hbm.at[h, c, pl.ds(r0, CHZ)])
        if h == 0:
            zero_acc()
            plsc.subcore_barrier()


_agg_kernel = functools.partial(
    pl.kernel,
    out_type=jax.ShapeDtypeStruct((2, NC, NP, DH), jnp.float32),
    mesh=_mesh,
    compiler_params=pltpu.CompilerParams(needs_layout_passes=False,
                                         use_tc_tiling_on_sc=False),
    scratch_types=[
        pltpu.VMEM((NCH, CH), jnp.int32),
        pltpu.VMEM((NCH, CH), jnp.int32),
        pltpu.VMEM((CH, DH), jnp.float32),
        pltpu.VMEM((CH, DH), jnp.float32),
        pltpu.VMEM((CH, DH), jnp.float32),
        pltpu.VMEM((CH, DH), jnp.float32),
        pltpu.VMEM((CHZ, DH), jnp.float32),
        pltpu.VMEM_SHARED((NP, DH), jnp.float32),
        pltpu.SemaphoreType.DMA,
        pltpu.SemaphoreType.DMA,
        pltpu.SemaphoreType.DMA,
        pltpu.SemaphoreType.DMA,
        pltpu.SemaphoreType.DMA,
        pltpu.SemaphoreType.DMA,
        pltpu.SemaphoreType.DMA,
        pltpu.SemaphoreType.DMA,
    ],
)(_agg_body)


def _norm(h_ref):
    deg = jnp.sum(h_ref[...], axis=(0, 1))
    return lax.rsqrt(jnp.clip(deg, 1.0, None))


def _l1_body(nid_ref, hs_ref, emb_ref, w0_ref, wr0_ref, br0_ref,
             ylo_ref, yhi_ref, res0_ref):
    nid = nid_ref[0, 0, :]
    onehot = (nid[:, None] ==
              lax.broadcasted_iota(jnp.int32, (BN, VPAD), 1)
              ).astype(jnp.float32)
    embf = emb_ref[...]
    t0 = _dot(embf, w0_ref[...])
    tr0 = jax.nn.relu(_dot(embf, wr0_ref[...]) + br0_ref[0:1, :])
    ns = _norm(hs_ref)
    y0 = _dot(onehot, t0) * ns[:, None]
    ylo_ref[...] = y0[:, :DH]
    yhi_ref[...] = y0[:, DH:]
    res0_ref[...] = _dot(onehot, tr0)


_hist_spec = pl.BlockSpec((NW, 1, BN), lambda i: (0, 0, i))


def _l1_call(nid3, hsp, emb_pad, w0, wr0, br0p):
    blk3 = pl.BlockSpec((1, 1, BN), lambda i: (i, 0, 0))
    vfull = pl.BlockSpec((VPAD, D), lambda i: (0, 0))
    wfull = pl.BlockSpec((D, D), lambda i: (0, 0))
    bfull = pl.BlockSpec((8, D), lambda i: (0, 0))
    rows = pl.BlockSpec((BN, D), lambda i: (i, 0))
    half = pl.BlockSpec((BN, DH), lambda i: (i, 0))
    return pl.pallas_call(
        _l1_body,
        grid=(G,),
        in_specs=[blk3, _hist_spec, vfull, wfull, wfull, bfull],
        out_specs=(half, half, rows),
        out_shape=(
            jax.ShapeDtypeStruct((NP, DH), jnp.float32),
            jax.ShapeDtypeStruct((NP, DH), jnp.float32),
            jax.ShapeDtypeStruct((NP, D), jnp.float32),
        ),
    )(nid3, hsp, emb_pad, w0, wr0, br0p)


def _agg_specs():
    return [pl.BlockSpec((1, 1, BN, DH), lambda i, h=h, c=c: (h, c, i, 0))
            for h in (0, 1) for c in (0, 1)]


def _combine(a_lo0, a_lo1, a_hi0, a_hi1):
    lo = a_lo0[0, 0] + a_lo1[0, 0]
    hi = a_hi0[0, 0] + a_hi1[0, 0]
    return jnp.concatenate([lo, hi], axis=-1)


def _mid_body(alo0_ref, alo1_ref, ahi0_ref, ahi1_ref, hs_ref, hd_ref,
              res0_ref, b0_ref, w1_ref, wr1_ref, br1_ref,
              ylo_ref, yhi_ref, res1_ref):
    ns = _norm(hs_ref)
    nd = _norm(hd_ref)
    agg = _combine(alo0_ref[...], alo1_ref[...],
                   ahi0_ref[...], ahi1_ref[...])
    h1 = jax.nn.relu(agg * nd[:, None] + b0_ref[0:1, :]) + res0_ref[...]
    y1 = _dot(h1, w1_ref[...]) * ns[:, None]
    ylo_ref[...] = y1[:, :DH]
    yhi_ref[...] = y1[:, DH:]
    res1_ref[...] = jax.nn.relu(_dot(h1, wr1_ref[...]) + br1_ref[0:1, :])


def _mid_call(aggout, hsp, hdp, res0, b0p, w1, wr1, br1p):
    s00, s01, s10, s11 = _agg_specs()
    rows = pl.BlockSpec((BN, D), lambda i: (i, 0))
    half = pl.BlockSpec((BN, DH), lambda i: (i, 0))
    wfull = pl.BlockSpec((D, D), lambda i: (0, 0))
    bfull = pl.BlockSpec((8, D), lambda i: (0, 0))
    return pl.pallas_call(
        _mid_body,
        grid=(G,),
        in_specs=[s00, s01, s10, s11, _hist_spec, _hist_spec, rows, bfull,
                  wfull, wfull, bfull],
        out_specs=(half, half, rows),
        out_shape=(
            jax.ShapeDtypeStruct((NP, DH), jnp.float32),
            jax.ShapeDtypeStruct((NP, DH), jnp.float32),
            jax.ShapeDtypeStruct((NP, D), jnp.float32),
        ),
    )(aggout, aggout, aggout, aggout, hsp, hdp, res0, b0p, w1, wr1, br1p)


def _fin_body(alo0_ref, alo1_ref, ahi0_ref, ahi1_ref, hd_ref, res1_ref,
              b1_ref, out_ref):
    nd = _norm(hd_ref)
    agg = _combine(alo0_ref[...], alo1_ref[...],
                   ahi0_ref[...], ahi1_ref[...])
    out_ref[...] = (jax.nn.relu(agg * nd[:, None] + b1_ref[0:1, :])
                    + res1_ref[...])


def _fin_call(aggout, hdp, res1, b1p):
    s00, s01, s10, s11 = _agg_specs()
    rows = pl.BlockSpec((BN, D), lambda i: (i, 0))
    bfull = pl.BlockSpec((8, D), lambda i: (0, 0))
    return pl.pallas_call(
        _fin_body,
        grid=(G,),
        in_specs=[s00, s01, s10, s11, _hist_spec, rows, bfull],
        out_specs=rows,
        out_shape=jax.ShapeDtypeStruct((NP, D), jnp.float32),
    )(aggout, aggout, aggout, aggout, hdp, res1, b1p)


def kernel(node_ids, edge_index, emb, W0, b0, Wr0, br0, W1, b1, Wr1, br1):
    src = edge_index[0].astype(jnp.int32)
    dst = edge_index[1].astype(jnp.int32)
    srcr = src.reshape(NW, NCH, CH)
    dstr = dst.reshape(NW, NCH, CH)
    srcf = src.reshape(NW, EPW)
    dstf = dst.reshape(NW, EPW)

    nid = node_ids.astype(jnp.int32)
    nid3 = jnp.pad(nid, (0, NP - N)).reshape(G, 1, BN)
    emb_pad = jnp.zeros((VPAD, D), jnp.float32).at[:VOCAB].set(emb)
    b0p = jnp.broadcast_to(b0[None, :], (8, D))
    br0p = jnp.broadcast_to(br0[None, :], (8, D))
    b1p = jnp.broadcast_to(b1[None, :], (8, D))
    br1p = jnp.broadcast_to(br1[None, :], (8, D))

    hs, hd = _deg_kernel(srcf, dstf)
    hsp = jnp.pad(hs, ((0, 0), (0, 0), (0, NP - N)))
    hdp = jnp.pad(hd, ((0, 0), (0, 0), (0, NP - N)))

    y0lo, y0hi, res0 = _l1_call(nid3, hsp, emb_pad, W0, Wr0, br0p)
    agg1 = _agg_kernel(y0lo, y0hi, srcr, dstr)
    y1lo, y1hi, res1 = _mid_call(agg1, hsp, hdp, res0, b0p, W1, Wr1, br1p)
    agg2 = _agg_kernel(y1lo, y1hi, srcr, dstr)
    h2 = _fin_call(agg2, hdp, res1, b1p)
    return h2[:N]
